# baseline (speedup 1.0000x reference)
import jax
import jax.numpy as jnp
from jax.experimental import pallas as pl
from jax.experimental.pallas import tpu as pltpu
from jax.experimental.pallas import tpu_sc as plsc

_TEMP = 0.05
_SHIFT = 1.0 / _TEMP
_LOG2E = 1.4426950408889634
_B = 1024
_D = 128
_N = 100000
_BLK = 25000
_GW = 128


def _stream_body(x_ref, f_ref, s_out_ref, xb_ref, s_ref):
    k = pl.program_id(0)
    nk = pl.num_programs(0)

    @pl.when(k == 0)
    def _():
        x = x_ref[...]
        nrm = jnp.sqrt(jnp.sum(x * x, axis=1, keepdims=True))
        xn = x / jnp.maximum(nrm, 1e-12)
        xb_ref[...] = (xn * (_SHIFT * _LOG2E)).astype(jnp.bfloat16)
        s_ref[...] = jnp.zeros((_B, 1), jnp.float32)

    l2 = jax.lax.dot_general(
        xb_ref[...], f_ref[...].astype(jnp.bfloat16),
        (((1,), (1,)), ((), ())),
        preferred_element_type=jnp.float32)
    s_ref[...] += jnp.sum(jnp.exp2(l2), axis=1, keepdims=True)

    @pl.when(k == nk - 1)
    def _():
        s_out_ref[...] = s_ref[...]


def _combine_body(s_ref, x_ref, g_ref, loss_ref):
    x = x_ref[...]
    nrm = jnp.sqrt(jnp.sum(x * x, axis=1, keepdims=True))
    xn = x / jnp.maximum(nrm, 1e-12)
    tgt = jnp.sum(xn * g_ref[...], axis=1, keepdims=True) * _SHIFT
    lse = jnp.log(s_ref[...])
    loss_ref[...] = jnp.mean(lse - tgt).reshape(1, 1)


def _sc_gather(features, t2d):
    mesh = plsc.VectorSubcoreMesh(core_axis_name="core",
                                  subcore_axis_name="subcore")

    @pl.kernel(out_type=jax.ShapeDtypeStruct((_B, _D), jnp.float32),
               mesh=mesh)
    def gather_kernel(f_hbm, i_hbm, o_hbm):
        def body(i_vmem, o_vmem):
            pltpu.sync_copy(f_hbm.at[i_vmem.at[0]], o_vmem)

        pltpu.emit_pipeline(
            body,
            grid=(_B // _GW,),
            in_specs=[pl.BlockSpec((1, _GW), index_map=lambda i: (0, i))],
            out_specs=[pl.BlockSpec((_GW, _D), index_map=lambda i: (i, 0))],
            core_axis_name=("core", "subcore"),
            dimension_semantics=(pltpu.PARALLEL,),
        )(i_hbm, o_hbm)

    return gather_kernel(features, t2d)


def kernel(inputs, targets, momentum, features):
    del momentum
    t2d = targets.astype(jnp.int32).reshape(1, _B)
    gathered = _sc_gather(features, t2d)

    s = pl.pallas_call(
        _stream_body,
        grid=(_N // _BLK,),
        in_specs=[
            pl.BlockSpec((_B, _D), lambda k: (0, 0)),
            pl.BlockSpec((_BLK, _D), lambda k: (k, 0)),
        ],
        out_specs=pl.BlockSpec((_B, 1), lambda k: (0, 0)),
        out_shape=jax.ShapeDtypeStruct((_B, 1), jnp.float32),
        scratch_shapes=[
            pltpu.VMEM((_B, _D), jnp.bfloat16),
            pltpu.VMEM((_B, 1), jnp.float32),
        ],
    )(inputs, features)

    loss = pl.pallas_call(
        _combine_body,
        out_shape=jax.ShapeDtypeStruct((1, 1), jnp.float32),
    )(s, inputs, gathered)
    return loss[0, 0]


# speedup vs baseline: 1.0030x; 1.0030x over previous
import jax
import jax.numpy as jnp
from jax.experimental import pallas as pl
from jax.experimental.pallas import tpu as pltpu
from jax.experimental.pallas import tpu_sc as plsc

_TEMP = 0.05
_SHIFT = 1.0 / _TEMP
_LOG2E = 1.4426950408889634
_B = 1024
_D = 128
_N = 100000
_BLK = 20000
_GW = 128


def _stream_body(x_ref, f_ref, s_out_ref, xb_ref, s_ref):
    k = pl.program_id(0)
    nk = pl.num_programs(0)

    @pl.when(k == 0)
    def _():
        x = x_ref[...]
        nrm = jnp.sqrt(jnp.sum(x * x, axis=1, keepdims=True))
        xn = x / jnp.maximum(nrm, 1e-12)
        xb_ref[...] = (xn * (_SHIFT * _LOG2E)).astype(jnp.bfloat16)
        s_ref[...] = jnp.zeros((_B, 1), jnp.float32)

    l2 = jax.lax.dot_general(
        xb_ref[...], f_ref[...].astype(jnp.bfloat16),
        (((1,), (1,)), ((), ())),
        preferred_element_type=jnp.float32)
    s_ref[...] += jnp.sum(jnp.exp2(l2), axis=1, keepdims=True)

    @pl.when(k == nk - 1)
    def _():
        s_out_ref[...] = s_ref[...]


def _combine_body(s_ref, x_ref, g_ref, loss_ref):
    x = x_ref[...]
    nrm = jnp.sqrt(jnp.sum(x * x, axis=1, keepdims=True))
    xn = x / jnp.maximum(nrm, 1e-12)
    tgt = jnp.sum(xn * g_ref[...], axis=1, keepdims=True) * _SHIFT
    lse = jnp.log(s_ref[...])
    loss_ref[...] = jnp.mean(lse - tgt).reshape(1, 1)


def _sc_gather(features, t2d):
    mesh = plsc.VectorSubcoreMesh(core_axis_name="core",
                                  subcore_axis_name="subcore")

    @pl.kernel(out_type=jax.ShapeDtypeStruct((_B, _D), jnp.float32),
               mesh=mesh)
    def gather_kernel(f_hbm, i_hbm, o_hbm):
        def body(i_vmem, o_vmem):
            pltpu.sync_copy(f_hbm.at[i_vmem.at[0]], o_vmem)

        pltpu.emit_pipeline(
            body,
            grid=(_B // _GW,),
            in_specs=[pl.BlockSpec((1, _GW), index_map=lambda i: (0, i))],
            out_specs=[pl.BlockSpec((_GW, _D), index_map=lambda i: (i, 0))],
            core_axis_name=("core", "subcore"),
            dimension_semantics=(pltpu.PARALLEL,),
        )(i_hbm, o_hbm)

    return gather_kernel(features, t2d)


def kernel(inputs, targets, momentum, features):
    del momentum
    t2d = targets.astype(jnp.int32).reshape(1, _B)
    gathered = _sc_gather(features, t2d)

    s = pl.pallas_call(
        _stream_body,
        grid=(_N // _BLK,),
        in_specs=[
            pl.BlockSpec((_B, _D), lambda k: (0, 0)),
            pl.BlockSpec((_BLK, _D), lambda k: (k, 0)),
        ],
        out_specs=pl.BlockSpec((_B, 1), lambda k: (0, 0)),
        out_shape=jax.ShapeDtypeStruct((_B, 1), jnp.float32),
        scratch_shapes=[
            pltpu.VMEM((_B, _D), jnp.bfloat16),
            pltpu.VMEM((_B, 1), jnp.float32),
        ],
    )(inputs, features)

    loss = pl.pallas_call(
        _combine_body,
        out_shape=jax.ShapeDtypeStruct((1, 1), jnp.float32),
    )(s, inputs, gathered)
    return loss[0, 0]
